# Initial kernel scaffold; baseline (speedup 1.0000x reference)
import jax
import jax.numpy as jnp
from jax.experimental import pallas as pl


def kernel(x, src, tgt, W_f, b_f, W_w, b_w):
    raise NotImplementedError("write your pallas kernel here")


# speedup vs baseline: 2.2288x; 2.2288x over previous
import functools

import jax
import jax.numpy as jnp
from jax import lax
from jax.experimental import pallas as pl
from jax.experimental.pallas import tpu as pltpu
from jax.experimental.pallas import tpu_sc as plsc

N = 10000
E = 160000
D = 256
H = 128
EPS = 1e-6

NC = 2
NS = 16
NW = NC * NS

K = 128
NB = E // K
NPAD = 10240

_mesh = plsc.VectorSubcoreMesh(core_axis_name="c", subcore_axis_name="s")
_sc_params = pltpu.CompilerParams(needs_layout_passes=False)


_BN = 1000


def _dense_body(x_ref, wf_ref, bf_ref, ww_ref, bw_ref,
                xs0_ref, xs1_ref, xt0_ref, xt1_ref, ws_ref, wt_ref):
    xb = x_ref[...]
    wf = wf_ref[...]
    dn = (((1,), (1,)), ((), ()))
    xs = lax.dot_general(xb, wf[:, :D], dn,
                         preferred_element_type=jnp.float32) + bf_ref[...]
    xt = lax.dot_general(xb, wf[:, D:], dn,
                         preferred_element_type=jnp.float32)
    xs0_ref[...] = xs[:, :H]
    xs1_ref[...] = xs[:, H:]
    xt0_ref[...] = xt[:, :H]
    xt1_ref[...] = xt[:, H:]
    ww = ww_ref[...]
    wsm = jnp.broadcast_to(ww[:, :D], (16, D))
    wtm = jnp.broadcast_to(ww[:, D:], (16, D))
    ws_ref[...] = lax.dot_general(xb, wsm, dn,
                                  preferred_element_type=jnp.float32)
    wt_ref[...] = lax.dot_general(xb, wtm, dn,
                                  preferred_element_type=jnp.float32) + bw_ref[0]


def _dense(x, W_f, b_f, W_w, b_w):
    nblk = N // _BN
    fs = jax.ShapeDtypeStruct
    return pl.pallas_call(
        _dense_body,
        grid=(nblk,),
        in_specs=[
            pl.BlockSpec((_BN, D), lambda i: (i, 0)),
            pl.BlockSpec((D, 2 * D), lambda i: (0, 0)),
            pl.BlockSpec((D,), lambda i: (0,)),
            pl.BlockSpec((1, 2 * D), lambda i: (0, 0)),
            pl.BlockSpec(memory_space=pltpu.MemorySpace.SMEM),
        ],
        out_specs=[
            pl.BlockSpec((_BN, H), lambda i: (i, 0)),
            pl.BlockSpec((_BN, H), lambda i: (i, 0)),
            pl.BlockSpec((_BN, H), lambda i: (i, 0)),
            pl.BlockSpec((_BN, H), lambda i: (i, 0)),
            pl.BlockSpec((_BN, 16), lambda i: (i, 0)),
            pl.BlockSpec((_BN, 16), lambda i: (i, 0)),
        ],
        out_shape=[
            fs((N, H), jnp.float32), fs((N, H), jnp.float32),
            fs((N, H), jnp.float32), fs((N, H), jnp.float32),
            fs((N, 16), jnp.float32), fs((N, 16), jnp.float32),
        ],
    )(x, W_f, b_f, W_w, b_w)


def _wid_partition(wid, nworkers):
    per = NB // nworkers
    extra = NB % nworkers
    nb = jnp.where(wid < extra, per + 1, per)
    base = per * wid + jnp.minimum(wid, extra)
    return nb, base


@functools.partial(
    pl.kernel,
    out_type=jax.ShapeDtypeStruct((NW, 16), jnp.float32),
    mesh=_mesh,
    compiler_params=_sc_params,
    scratch_types=[
        pltpu.VMEM((K,), jnp.int32),
        pltpu.VMEM((K,), jnp.int32),
        pltpu.VMEM((K,), jnp.float32),
        pltpu.VMEM((K,), jnp.float32),
        pltpu.VMEM((16,), jnp.float32),
    ],
)
def _edge_max(src_hbm, tgt_hbm, ws_hbm, wt_hbm, mx_hbm,
              sidx, tidx, wsr, wtr, mxv):
    c = lax.axis_index("c")
    s = lax.axis_index("s")
    wid = s * NC + c
    nb, base = _wid_partition(wid, NW)
    mxv[...] = jnp.full((16,), -3e38, jnp.float32)

    def batch(b, _):
        off = (base + b) * K
        pltpu.sync_copy(src_hbm.at[pl.ds(off, K)], sidx)
        pltpu.sync_copy(tgt_hbm.at[pl.ds(off, K)], tidx)
        pltpu.sync_copy(ws_hbm.at[sidx], wsr)
        pltpu.sync_copy(wt_hbm.at[tidx], wtr)

        def edge(k, _):
            a = wsr[pl.ds(k * 16, 16)] + wtr[pl.ds(k * 16, 16)]
            mxv[...] = jnp.maximum(mxv[...], a)
            return 0

        lax.fori_loop(0, K // 16, edge, 0)
        return 0

    lax.fori_loop(0, nb, batch, 0)
    pltpu.sync_copy(mxv, mx_hbm.at[wid])


@functools.partial(
    pl.kernel,
    out_type=jax.ShapeDtypeStruct((NC, N, H), jnp.float32),
    mesh=_mesh,
    compiler_params=_sc_params,
    scratch_types=[
        pltpu.VMEM((K,), jnp.int32),
        pltpu.VMEM((K,), jnp.int32),
        pltpu.VMEM((K, H), jnp.float32),
        pltpu.VMEM((K, H), jnp.float32),
        pltpu.VMEM((K,), jnp.float32),
        pltpu.VMEM((K,), jnp.float32),
        pltpu.VMEM((K,), jnp.float32),
        pltpu.VMEM((NW, 16), jnp.float32),
        pltpu.VMEM((16,), jnp.float32),
        pltpu.VMEM_SHARED((NPAD, H), jnp.float32),
        pltpu.VMEM_SHARED((NPAD,), jnp.float32),
    ],
)
def _aggregate(src_hbm, tgt_hbm, xs0_hbm, xs1_hbm, xt0_hbm, xt1_hbm,
               ws_hbm, wt_hbm, mx_hbm, o_hbm,
               sidx, tidx, rs, rt, wsr, wtr, aexp, mxl, mg, acc, den):
    c = lax.axis_index("c")
    s = lax.axis_index("s")
    nb, base = _wid_partition(s, NS)

    pltpu.sync_copy(mx_hbm, mxl)
    mxv_init = jnp.full((16,), -3e38, jnp.float32)

    def mx_red(i, m):
        return jnp.maximum(m, mxl[i, :])

    gmaxv = lax.fori_loop(0, NW, mx_red, mxv_init)
    mg[...] = gmaxv
    lanes = lax.iota(jnp.int32, 16)
    for shift in (1, 2, 4, 8):
        other = plsc.load_gather(mg, [lanes ^ shift])
        mg[...] = jnp.maximum(mg[...], other)
    gmax = mg[...]

    def zrow(i, _):
        for cc in range(H // 16):
            rs[i, pl.ds(cc * 16, 16)] = jnp.zeros((16,), jnp.float32)
        return 0

    lax.fori_loop(0, K, zrow, 0)

    def zae(i, _):
        aexp[pl.ds(i * 16, 16)] = jnp.zeros((16,), jnp.float32)
        return 0

    lax.fori_loop(0, K // 16, zae, 0)
    for j in range(5):
        row0 = s * 640 + j * K
        pltpu.sync_copy(rs, acc.at[pl.ds(row0, K)])
        pltpu.sync_copy(aexp, den.at[pl.ds(row0, K)])
    plsc.subcore_barrier()

    def run(xs_hbm, xt_hbm):
        def batch(b, _):
            off = (base + b) * K
            pltpu.sync_copy(src_hbm.at[pl.ds(off, K)], sidx)
            pltpu.sync_copy(tgt_hbm.at[pl.ds(off, K)], tidx)
            pltpu.sync_copy(xs_hbm.at[sidx], rs)
            pltpu.sync_copy(xt_hbm.at[tidx], rt)
            pltpu.sync_copy(ws_hbm.at[sidx], wsr)
            pltpu.sync_copy(wt_hbm.at[tidx], wtr)

            def ae16(k, _):
                a = wsr[pl.ds(k * 16, 16)] + wtr[pl.ds(k * 16, 16)]
                aexp[pl.ds(k * 16, 16)] = jnp.exp(a - gmax)
                return 0

            lax.fori_loop(0, K // 16, ae16, 0)

            def edge16(g, _):
                ae16 = aexp[pl.ds(g * 16, 16)]
                for j in range(16):
                    k = g * 16 + j
                    ae = ae16[j]
                    for cc in range(H // 16):
                        v = (rs[k, pl.ds(cc * 16, 16)]
                             + rt[k, pl.ds(cc * 16, 16)])
                        rs[k, pl.ds(cc * 16, 16)] = (
                            jnp.maximum(v, 0.0) * ae)
                return 0

            lax.fori_loop(0, K // 16, edge16, 0)
            pltpu.sync_copy(rs, acc.at[tidx], add=True)
            pltpu.sync_copy(aexp, den.at[tidx], add=True)
            return 0

        lax.fori_loop(0, nb, batch, 0)

    @pl.when(c == 0)
    def _():
        run(xs0_hbm, xt0_hbm)

    @pl.when(c == 1)
    def _():
        run(xs1_hbm, xt1_hbm)

    plsc.subcore_barrier()

    for j in range(5):
        row0 = s * 640 + j * K
        pltpu.sync_copy(acc.at[pl.ds(row0, K)], rs)
        pltpu.sync_copy(den.at[pl.ds(row0, K)], wsr)

        def node16(g, _):
            r16 = 1.0 / (wsr[pl.ds(g * 16, 16)] + EPS)
            for j in range(16):
                n = g * 16 + j
                r = r16[j]
                for cc in range(H // 16):
                    rs[n, pl.ds(cc * 16, 16)] = rs[n, pl.ds(cc * 16, 16)] * r
            return 0

        lax.fori_loop(0, K // 16, node16, 0)
        full = row0 + K <= N

        @pl.when(full)
        def _():
            pltpu.sync_copy(rs, o_hbm.at[c, pl.ds(row0, K)])

        @pl.when(jnp.logical_and(jnp.logical_not(full), row0 < N))
        def _():
            pltpu.sync_copy(rs.at[pl.ds(0, 16)], o_hbm.at[c, pl.ds(row0, 16)])


def kernel(x, src, tgt, W_f, b_f, W_w, b_w):
    src = src.astype(jnp.int32)
    tgt = tgt.astype(jnp.int32)
    W_w2 = W_w.reshape(1, 2 * D)
    xs0, xs1, xt0, xt1, ws2, wt2 = _dense(x, W_f, b_f, W_w2, b_w)
    ws1 = ws2[:, 0]
    wt1 = wt2[:, 0]
    mx = _edge_max(src, tgt, ws1, wt1)
    o2 = _aggregate(src, tgt, xs0, xs1, xt0, xt1, ws1, wt1, mx)
    return jnp.concatenate([o2[0], o2[1]], axis=1)


# speedup vs baseline: 3.8201x; 1.7140x over previous
import functools

import jax
import jax.numpy as jnp
from jax import lax
from jax.experimental import pallas as pl
from jax.experimental.pallas import tpu as pltpu
from jax.experimental.pallas import tpu_sc as plsc

N = 10000
E = 160000
D = 256
H = 128
EPS = 1e-6

NC = 2
NS = 16
NW = NC * NS

K = 128
K2 = 64
NPAD = 10240

_mesh = plsc.VectorSubcoreMesh(core_axis_name="c", subcore_axis_name="s")
_sc_params = pltpu.CompilerParams(needs_layout_passes=False)


_BN = 1000


def _dense_body(x_ref, wf_ref, bf_ref, ww_ref, bw_ref,
                xs0_ref, xs1_ref, xt0_ref, xt1_ref, ws_ref, wt_ref):
    xb = x_ref[...]
    wf = wf_ref[...]
    dn = (((1,), (1,)), ((), ()))
    xs = lax.dot_general(xb, wf[:, :D], dn,
                         preferred_element_type=jnp.float32) + bf_ref[...]
    xt = lax.dot_general(xb, wf[:, D:], dn,
                         preferred_element_type=jnp.float32)
    xs0_ref[...] = xs[:, :H]
    xs1_ref[...] = xs[:, H:]
    xt0_ref[...] = xt[:, :H]
    xt1_ref[...] = xt[:, H:]
    ww = ww_ref[...]
    wsm = jnp.broadcast_to(ww[:, :D], (16, D))
    wtm = jnp.broadcast_to(ww[:, D:], (16, D))
    ws_ref[...] = lax.dot_general(xb, wsm, dn,
                                  preferred_element_type=jnp.float32)
    wt_ref[...] = lax.dot_general(xb, wtm, dn,
                                  preferred_element_type=jnp.float32) + bw_ref[0]


def _dense(x, W_f, b_f, W_w, b_w):
    nblk = N // _BN
    fs = jax.ShapeDtypeStruct
    return pl.pallas_call(
        _dense_body,
        grid=(nblk,),
        in_specs=[
            pl.BlockSpec((_BN, D), lambda i: (i, 0)),
            pl.BlockSpec((D, 2 * D), lambda i: (0, 0)),
            pl.BlockSpec((D,), lambda i: (0,)),
            pl.BlockSpec((1, 2 * D), lambda i: (0, 0)),
            pl.BlockSpec(memory_space=pltpu.MemorySpace.SMEM),
        ],
        out_specs=[
            pl.BlockSpec((_BN, H), lambda i: (i, 0)),
            pl.BlockSpec((_BN, H), lambda i: (i, 0)),
            pl.BlockSpec((_BN, H), lambda i: (i, 0)),
            pl.BlockSpec((_BN, H), lambda i: (i, 0)),
            pl.BlockSpec((_BN, 16), lambda i: (i, 0)),
            pl.BlockSpec((_BN, 16), lambda i: (i, 0)),
        ],
        out_shape=[
            fs((N, H), jnp.float32), fs((N, H), jnp.float32),
            fs((N, H), jnp.float32), fs((N, H), jnp.float32),
            fs((N, 16), jnp.float32), fs((N, 16), jnp.float32),
        ],
    )(x, W_f, b_f, W_w, b_w)


def _partition(wid, nbatches, nworkers):
    per = nbatches // nworkers
    extra = nbatches % nworkers
    nb = jnp.where(wid < extra, per + 1, per)
    base = per * wid + jnp.minimum(wid, extra)
    return nb, base


@functools.partial(
    pl.kernel,
    out_type=[jax.ShapeDtypeStruct((E,), jnp.float32),
              jax.ShapeDtypeStruct((NW, 16), jnp.float32)],
    mesh=_mesh,
    compiler_params=_sc_params,
    scratch_types=[
        pltpu.VMEM((2, K), jnp.int32),
        pltpu.VMEM((2, K), jnp.int32),
        pltpu.VMEM((2, K), jnp.float32),
        pltpu.VMEM((2, K), jnp.float32),
        pltpu.VMEM((2, K), jnp.float32),
        pltpu.VMEM((16,), jnp.float32),
        pltpu.SemaphoreType.DMA((2,)),
        pltpu.SemaphoreType.DMA((2,)),
    ],
)
def _edge_logits(src_hbm, tgt_hbm, ws_hbm, wt_hbm, a_hbm, mx_hbm,
                 sb, tb, wsb, wtb, ao, mxv, gsem, ssem):
    c = lax.axis_index("c")
    s = lax.axis_index("s")
    wid = s * NC + c
    nb, base = _partition(wid, E // K, NW)
    mxv[...] = jnp.full((16,), -3e38, jnp.float32)

    def fire(b, slot):
        off = (base + b) * K
        pltpu.sync_copy(src_hbm.at[pl.ds(off, K)], sb.at[slot])
        pltpu.sync_copy(tgt_hbm.at[pl.ds(off, K)], tb.at[slot])
        pltpu.async_copy(ws_hbm.at[sb.at[slot]], wsb.at[slot], gsem.at[slot])
        pltpu.async_copy(wt_hbm.at[tb.at[slot]], wtb.at[slot], gsem.at[slot])

    def drain_g(slot):
        pltpu.make_async_copy(ws_hbm.at[sb.at[slot]], wsb.at[slot],
                              gsem.at[slot]).wait()
        pltpu.make_async_copy(wt_hbm.at[tb.at[slot]], wtb.at[slot],
                              gsem.at[slot]).wait()

    def drain_s(slot):
        pltpu.make_async_copy(ao.at[slot], a_hbm.at[pl.ds(0, K)],
                              ssem.at[slot]).wait()

    fire(0, 0)

    def phase(b, _):
        cur = b % 2
        oth = 1 - cur
        pl.when(b >= 1)(lambda: drain_s(oth))
        pl.when(b + 1 < nb)(lambda: fire(b + 1, oth))
        drain_g(cur)

        def grp(g, _):
            a16 = wsb[cur, pl.ds(g * 16, 16)] + wtb[cur, pl.ds(g * 16, 16)]
            ao[cur, pl.ds(g * 16, 16)] = a16
            mxv[...] = jnp.maximum(mxv[...], a16)
            return 0

        lax.fori_loop(0, K // 16, grp, 0)
        off = (base + b) * K
        pltpu.async_copy(ao.at[cur], a_hbm.at[pl.ds(off, K)], ssem.at[cur])
        return 0

    lax.fori_loop(0, nb, phase, 0)
    pl.when(nb % 2 == 1)(lambda: drain_s(0))
    pl.when(nb % 2 == 0)(lambda: drain_s(1))
    pltpu.sync_copy(mxv, mx_hbm.at[wid])


@functools.partial(
    pl.kernel,
    out_type=jax.ShapeDtypeStruct((NC, N, H), jnp.float32),
    mesh=_mesh,
    compiler_params=_sc_params,
    scratch_types=[
        pltpu.VMEM((2, K2), jnp.int32),
        pltpu.VMEM((2, K2), jnp.int32),
        pltpu.VMEM((2, K2), jnp.float32),
        pltpu.VMEM((2, K2, H), jnp.float32),
        pltpu.VMEM((2, K2, H), jnp.float32),
        pltpu.VMEM((2, K2), jnp.float32),
        pltpu.VMEM((NW, 16), jnp.float32),
        pltpu.VMEM((16,), jnp.float32),
        pltpu.SemaphoreType.DMA((2,)),
        pltpu.SemaphoreType.DMA((2,)),
        pltpu.VMEM_SHARED((NPAD, H), jnp.float32),
        pltpu.VMEM_SHARED((NPAD,), jnp.float32),
    ],
)
def _aggregate(src_hbm, tgt_hbm, a_hbm, xs0_hbm, xs1_hbm, xt0_hbm, xt1_hbm,
               mx_hbm, o_hbm,
               sb2, tb2, ab, rs, rt, aeb, mxl, mg, gsem, ssem, acc, den):
    c = lax.axis_index("c")
    s = lax.axis_index("s")
    nb, base = _partition(s, E // K2, NS)

    pltpu.sync_copy(mx_hbm, mxl)
    mxv_init = jnp.full((16,), -3e38, jnp.float32)

    def mx_red(i, m):
        return jnp.maximum(m, mxl[i, :])

    gmaxv = lax.fori_loop(0, NW, mx_red, mxv_init)
    mg[...] = gmaxv
    lanes = lax.iota(jnp.int32, 16)
    for shift in (1, 2, 4, 8):
        other = plsc.load_gather(mg, [lanes ^ shift])
        mg[...] = jnp.maximum(mg[...], other)
    gmax = mg[...]

    def zrow(i, _):
        for cc in range(H // 16):
            rs[0, i, pl.ds(cc * 16, 16)] = jnp.zeros((16,), jnp.float32)
        return 0

    lax.fori_loop(0, K2, zrow, 0)
    for g in range(K2 // 16):
        ab[0, pl.ds(g * 16, 16)] = jnp.zeros((16,), jnp.float32)
    for j in range(10):
        row0 = s * 640 + j * K2
        pltpu.sync_copy(rs.at[0], acc.at[pl.ds(row0, K2)])
        pltpu.sync_copy(ab.at[0], den.at[pl.ds(row0, K2)])
    plsc.subcore_barrier()

    def run(xs_hbm, xt_hbm):
        def fire(b, slot):
            off = (base + b) * K2
            pltpu.sync_copy(src_hbm.at[pl.ds(off, K2)], sb2.at[slot])
            pltpu.sync_copy(tgt_hbm.at[pl.ds(off, K2)], tb2.at[slot])
            pltpu.sync_copy(a_hbm.at[pl.ds(off, K2)], ab.at[slot])
            pltpu.async_copy(xs_hbm.at[sb2.at[slot]], rs.at[slot],
                             gsem.at[slot])
            pltpu.async_copy(xt_hbm.at[tb2.at[slot]], rt.at[slot],
                             gsem.at[slot])

        def drain_g(slot):
            pltpu.make_async_copy(xs_hbm.at[sb2.at[slot]], rs.at[slot],
                                  gsem.at[slot]).wait()
            pltpu.make_async_copy(xt_hbm.at[tb2.at[slot]], rt.at[slot],
                                  gsem.at[slot]).wait()

        def drain_s(slot):
            pltpu.make_async_copy(rs.at[slot], acc.at[pl.ds(0, K2)],
                                  ssem.at[slot]).wait()
            pltpu.make_async_copy(aeb.at[slot], den.at[pl.ds(0, K2)],
                                  ssem.at[slot]).wait()

        fire(0, 0)

        def phase(b, _):
            cur = b % 2
            oth = 1 - cur
            pl.when(b >= 1)(lambda: drain_s(oth))
            pl.when(b + 1 < nb)(lambda: fire(b + 1, oth))
            drain_g(cur)

            def grp(g, _):
                ae16 = jnp.exp(ab[cur, pl.ds(g * 16, 16)] - gmax)
                aeb[cur, pl.ds(g * 16, 16)] = ae16
                for j in range(16):
                    k = g * 16 + j
                    ae = ae16[j]
                    for cc in range(H // 16):
                        v = (rs[cur, k, pl.ds(cc * 16, 16)]
                             + rt[cur, k, pl.ds(cc * 16, 16)])
                        rs[cur, k, pl.ds(cc * 16, 16)] = (
                            jnp.maximum(v, 0.0) * ae)
                return 0

            lax.fori_loop(0, K2 // 16, grp, 0)
            pltpu.async_copy(rs.at[cur], acc.at[tb2.at[cur]],
                             ssem.at[cur], add=True)
            pltpu.async_copy(aeb.at[cur], den.at[tb2.at[cur]],
                             ssem.at[cur], add=True)
            return 0

        lax.fori_loop(0, nb, phase, 0)
        pl.when(nb % 2 == 1)(lambda: drain_s(0))
        pl.when(nb % 2 == 0)(lambda: drain_s(1))

    @pl.when(c == 0)
    def _():
        run(xs0_hbm, xt0_hbm)

    @pl.when(c == 1)
    def _():
        run(xs1_hbm, xt1_hbm)

    plsc.subcore_barrier()

    for j in range(10):
        row0 = s * 640 + j * K2
        pltpu.sync_copy(acc.at[pl.ds(row0, K2)], rs.at[0])
        pltpu.sync_copy(den.at[pl.ds(row0, K2)], ab.at[0])

        def node16(g, _):
            r16 = 1.0 / (ab[0, pl.ds(g * 16, 16)] + EPS)
            for j16 in range(16):
                n = g * 16 + j16
                r = r16[j16]
                for cc in range(H // 16):
                    rs[0, n, pl.ds(cc * 16, 16)] = (
                        rs[0, n, pl.ds(cc * 16, 16)] * r)
            return 0

        lax.fori_loop(0, K2 // 16, node16, 0)
        full = row0 + K2 <= N

        @pl.when(full)
        def _():
            pltpu.sync_copy(rs.at[0], o_hbm.at[c, pl.ds(row0, K2)])

        @pl.when(jnp.logical_and(jnp.logical_not(full), row0 < N))
        def _():
            pltpu.sync_copy(rs.at[0, pl.ds(0, 16)],
                            o_hbm.at[c, pl.ds(row0, 16)])


def kernel(x, src, tgt, W_f, b_f, W_w, b_w):
    src = src.astype(jnp.int32)
    tgt = tgt.astype(jnp.int32)
    W_w2 = W_w.reshape(1, 2 * D)
    xs0, xs1, xt0, xt1, ws2, wt2 = _dense(x, W_f, b_f, W_w2, b_w)
    ws1 = ws2[:, 0]
    wt1 = wt2[:, 0]
    a_e, mx = _edge_logits(src, tgt, ws1, wt1)
    o2 = _aggregate(src, tgt, a_e, xs0, xs1, xt0, xt1, mx)
    return jnp.concatenate([o2[0], o2[1]], axis=1)


# speedup vs baseline: 5.9691x; 1.5625x over previous
import functools

import jax
import jax.numpy as jnp
from jax import lax
from jax.experimental import pallas as pl
from jax.experimental.pallas import tpu as pltpu
from jax.experimental.pallas import tpu_sc as plsc

N = 10000
E = 160000
D = 256
H = 128
EPS = 1e-6

NC = 2
NS = 16
NW = NC * NS

K = 128
K2 = 64
NPAD = 10240

_mesh = plsc.VectorSubcoreMesh(core_axis_name="c", subcore_axis_name="s")
_sc_params = pltpu.CompilerParams(needs_layout_passes=False)


_BN = 1000


def _dense_body(x_ref, wf_ref, bf_ref, ww_ref, bw_ref,
                xs0_ref, xs1_ref, xt0_ref, xt1_ref, ws_ref, wt_ref):
    xb = x_ref[...]
    wf = wf_ref[...]
    dn = (((1,), (1,)), ((), ()))
    xs = lax.dot_general(xb, wf[:, :D], dn,
                         preferred_element_type=jnp.float32) + bf_ref[...]
    xt = lax.dot_general(xb, wf[:, D:], dn,
                         preferred_element_type=jnp.float32)
    xs0_ref[...] = xs[:, :H]
    xs1_ref[...] = xs[:, H:]
    xt0_ref[...] = xt[:, :H]
    xt1_ref[...] = xt[:, H:]
    ww = ww_ref[...]
    wsm = jnp.broadcast_to(ww[:, :D], (16, D))
    wtm = jnp.broadcast_to(ww[:, D:], (16, D))
    ws_ref[...] = lax.dot_general(xb, wsm, dn,
                                  preferred_element_type=jnp.float32)
    wt_ref[...] = lax.dot_general(xb, wtm, dn,
                                  preferred_element_type=jnp.float32) + bw_ref[0]


def _dense(x, W_f, b_f, W_w, b_w):
    nblk = N // _BN
    fs = jax.ShapeDtypeStruct
    return pl.pallas_call(
        _dense_body,
        grid=(nblk,),
        in_specs=[
            pl.BlockSpec((_BN, D), lambda i: (i, 0)),
            pl.BlockSpec((D, 2 * D), lambda i: (0, 0)),
            pl.BlockSpec((D,), lambda i: (0,)),
            pl.BlockSpec((1, 2 * D), lambda i: (0, 0)),
            pl.BlockSpec(memory_space=pltpu.MemorySpace.SMEM),
        ],
        out_specs=[
            pl.BlockSpec((_BN, H), lambda i: (i, 0)),
            pl.BlockSpec((_BN, H), lambda i: (i, 0)),
            pl.BlockSpec((_BN, H), lambda i: (i, 0)),
            pl.BlockSpec((_BN, H), lambda i: (i, 0)),
            pl.BlockSpec((_BN, 16), lambda i: (i, 0)),
            pl.BlockSpec((_BN, 16), lambda i: (i, 0)),
        ],
        out_shape=[
            fs((N, H), jnp.float32), fs((N, H), jnp.float32),
            fs((N, H), jnp.float32), fs((N, H), jnp.float32),
            fs((N, 16), jnp.float32), fs((N, 16), jnp.float32),
        ],
    )(x, W_f, b_f, W_w, b_w)


def _partition(wid, nbatches, nworkers):
    per = nbatches // nworkers
    extra = nbatches % nworkers
    nb = jnp.where(wid < extra, per + 1, per)
    base = per * wid + jnp.minimum(wid, extra)
    return nb, base


@functools.partial(
    pl.kernel,
    out_type=[jax.ShapeDtypeStruct((E,), jnp.float32),
              jax.ShapeDtypeStruct((NW, 16), jnp.float32)],
    mesh=_mesh,
    compiler_params=_sc_params,
    scratch_types=[
        pltpu.VMEM((2, K), jnp.int32),
        pltpu.VMEM((2, K), jnp.int32),
        pltpu.VMEM((2, K), jnp.float32),
        pltpu.VMEM((2, K), jnp.float32),
        pltpu.VMEM((2, K), jnp.float32),
        pltpu.VMEM((16,), jnp.float32),
        pltpu.SemaphoreType.DMA((2,)),
        pltpu.SemaphoreType.DMA((2,)),
    ],
)
def _edge_logits(src_hbm, tgt_hbm, ws_hbm, wt_hbm, a_hbm, mx_hbm,
                 sb, tb, wsb, wtb, ao, mxv, gsem, ssem):
    c = lax.axis_index("c")
    s = lax.axis_index("s")
    wid = s * NC + c
    nb, base = _partition(wid, E // K, NW)
    mxv[...] = jnp.full((16,), -3e38, jnp.float32)

    def fire(b, slot):
        off = (base + b) * K
        pltpu.sync_copy(src_hbm.at[pl.ds(off, K)], sb.at[slot])
        pltpu.sync_copy(tgt_hbm.at[pl.ds(off, K)], tb.at[slot])
        pltpu.async_copy(ws_hbm.at[sb.at[slot]], wsb.at[slot], gsem.at[slot])
        pltpu.async_copy(wt_hbm.at[tb.at[slot]], wtb.at[slot], gsem.at[slot])

    def drain_g(slot):
        pltpu.make_async_copy(ws_hbm.at[sb.at[slot]], wsb.at[slot],
                              gsem.at[slot]).wait()
        pltpu.make_async_copy(wt_hbm.at[tb.at[slot]], wtb.at[slot],
                              gsem.at[slot]).wait()

    def drain_s(slot):
        pltpu.make_async_copy(ao.at[slot], a_hbm.at[pl.ds(0, K)],
                              ssem.at[slot]).wait()

    fire(0, 0)

    def phase(b, _):
        cur = b % 2
        oth = 1 - cur
        pl.when(b >= 1)(lambda: drain_s(oth))
        pl.when(b + 1 < nb)(lambda: fire(b + 1, oth))
        drain_g(cur)

        def grp(g, _):
            a16 = wsb[cur, pl.ds(g * 16, 16)] + wtb[cur, pl.ds(g * 16, 16)]
            ao[cur, pl.ds(g * 16, 16)] = a16
            mxv[...] = jnp.maximum(mxv[...], a16)
            return 0

        lax.fori_loop(0, K // 16, grp, 0)
        off = (base + b) * K
        pltpu.async_copy(ao.at[cur], a_hbm.at[pl.ds(off, K)], ssem.at[cur])
        return 0

    lax.fori_loop(0, nb, phase, 0)
    pl.when(nb % 2 == 1)(lambda: drain_s(0))
    pl.when(nb % 2 == 0)(lambda: drain_s(1))
    pltpu.sync_copy(mxv, mx_hbm.at[wid])


@functools.partial(
    pl.kernel,
    out_type=jax.ShapeDtypeStruct((NC, N, H), jnp.float32),
    mesh=_mesh,
    compiler_params=_sc_params,
    scratch_types=[
        pltpu.VMEM((3, K2), jnp.int32),
        pltpu.VMEM((3, K2), jnp.int32),
        pltpu.VMEM((3, K2), jnp.float32),
        pltpu.VMEM((2, K2, H), jnp.float32),
        pltpu.VMEM((2, K2, H), jnp.float32),
        pltpu.VMEM((2, K2), jnp.float32),
        pltpu.VMEM((NW, 16), jnp.float32),
        pltpu.VMEM((16,), jnp.float32),
        pltpu.SemaphoreType.DMA((3,)),
        pltpu.SemaphoreType.DMA((2,)),
        pltpu.SemaphoreType.DMA((2,)),
        pltpu.VMEM_SHARED((NPAD, H), jnp.float32),
        pltpu.VMEM_SHARED((NPAD,), jnp.float32),
    ],
)
def _aggregate(src_hbm, tgt_hbm, a_hbm, xs0_hbm, xs1_hbm, xt0_hbm, xt1_hbm,
               mx_hbm, o_hbm,
               sb2, tb2, ab, rs, rt, aeb, mxl, mg, isem, gsem, ssem, acc, den):
    c = lax.axis_index("c")
    s = lax.axis_index("s")
    nb, base = _partition(s, E // K2, NS)

    pltpu.sync_copy(mx_hbm, mxl)
    mxv_init = jnp.full((16,), -3e38, jnp.float32)

    def mx_red(i, m):
        return jnp.maximum(m, mxl[i, :])

    gmaxv = lax.fori_loop(0, NW, mx_red, mxv_init)
    mg[...] = gmaxv
    lanes = lax.iota(jnp.int32, 16)
    for shift in (1, 2, 4, 8):
        other = plsc.load_gather(mg, [lanes ^ shift])
        mg[...] = jnp.maximum(mg[...], other)
    gmax = mg[...]

    def zrow(i, _):
        for cc in range(H // 16):
            rs[0, i, pl.ds(cc * 16, 16)] = jnp.zeros((16,), jnp.float32)
        return 0

    lax.fori_loop(0, K2, zrow, 0)
    for g in range(K2 // 16):
        ab[0, pl.ds(g * 16, 16)] = jnp.zeros((16,), jnp.float32)
    for j in range(10):
        row0 = s * 640 + j * K2
        pltpu.sync_copy(rs.at[0], acc.at[pl.ds(row0, K2)])
        pltpu.sync_copy(ab.at[0], den.at[pl.ds(row0, K2)])
    plsc.subcore_barrier()

    def run(xs_hbm, xt_hbm):
        def fire_idx(b, s3):
            off = (base + b) * K2
            pltpu.async_copy(src_hbm.at[pl.ds(off, K2)], sb2.at[s3],
                             isem.at[s3])
            pltpu.async_copy(tgt_hbm.at[pl.ds(off, K2)], tb2.at[s3],
                             isem.at[s3])
            pltpu.async_copy(a_hbm.at[pl.ds(off, K2)], ab.at[s3],
                             isem.at[s3])

        def drain_idx(s3):
            pltpu.make_async_copy(src_hbm.at[pl.ds(0, K2)], sb2.at[s3],
                                  isem.at[s3]).wait()
            pltpu.make_async_copy(tgt_hbm.at[pl.ds(0, K2)], tb2.at[s3],
                                  isem.at[s3]).wait()
            pltpu.make_async_copy(a_hbm.at[pl.ds(0, K2)], ab.at[s3],
                                  isem.at[s3]).wait()

        def fire_g(b, slot):
            s3 = b % 3
            pltpu.async_copy(xs_hbm.at[sb2.at[s3]], rs.at[slot],
                             gsem.at[slot])
            pltpu.async_copy(xt_hbm.at[tb2.at[s3]], rt.at[slot],
                             gsem.at[slot])

        def drain_g(slot):
            pltpu.make_async_copy(xs_hbm.at[sb2.at[0]], rs.at[slot],
                                  gsem.at[slot]).wait()
            pltpu.make_async_copy(xt_hbm.at[tb2.at[0]], rt.at[slot],
                                  gsem.at[slot]).wait()

        def drain_s(slot):
            pltpu.make_async_copy(rs.at[slot], acc.at[pl.ds(0, K2)],
                                  ssem.at[slot]).wait()
            pltpu.make_async_copy(aeb.at[slot], den.at[pl.ds(0, K2)],
                                  ssem.at[slot]).wait()

        fire_idx(0, 0)
        pl.when(nb >= 2)(lambda: fire_idx(1, 1))
        drain_idx(0)
        fire_g(0, 0)

        def phase(b, _):
            cur = b % 2
            oth = 1 - cur
            c3 = b % 3
            pl.when(b >= 1)(lambda: drain_s(oth))
            pl.when(b + 2 < nb)(lambda: fire_idx(b + 2, (b + 2) % 3))

            def prep_next():
                drain_idx((b + 1) % 3)
                fire_g(b + 1, oth)

            pl.when(b + 1 < nb)(prep_next)
            drain_g(cur)

            def grp(g, _):
                ae16 = jnp.exp(ab[c3, pl.ds(g * 16, 16)] - gmax)
                aeb[cur, pl.ds(g * 16, 16)] = ae16
                for j in range(16):
                    k = g * 16 + j
                    ae = ae16[j]
                    for cc in range(H // 16):
                        v = (rs[cur, k, pl.ds(cc * 16, 16)]
                             + rt[cur, k, pl.ds(cc * 16, 16)])
                        rs[cur, k, pl.ds(cc * 16, 16)] = (
                            jnp.maximum(v, 0.0) * ae)
                return 0

            lax.fori_loop(0, K2 // 16, grp, 0)
            pltpu.async_copy(rs.at[cur], acc.at[tb2.at[c3]],
                             ssem.at[cur], add=True)
            pltpu.async_copy(aeb.at[cur], den.at[tb2.at[c3]],
                             ssem.at[cur], add=True)
            return 0

        lax.fori_loop(0, nb, phase, 0)
        pl.when(nb % 2 == 1)(lambda: drain_s(0))
        pl.when(nb % 2 == 0)(lambda: drain_s(1))

    @pl.when(c == 0)
    def _():
        run(xs0_hbm, xt0_hbm)

    @pl.when(c == 1)
    def _():
        run(xs1_hbm, xt1_hbm)

    plsc.subcore_barrier()

    for j in range(10):
        row0 = s * 640 + j * K2
        pltpu.sync_copy(acc.at[pl.ds(row0, K2)], rs.at[0])
        pltpu.sync_copy(den.at[pl.ds(row0, K2)], ab.at[0])

        def node16(g, _):
            r16 = 1.0 / (ab[0, pl.ds(g * 16, 16)] + EPS)
            for j16 in range(16):
                n = g * 16 + j16
                r = r16[j16]
                for cc in range(H // 16):
                    rs[0, n, pl.ds(cc * 16, 16)] = (
                        rs[0, n, pl.ds(cc * 16, 16)] * r)
            return 0

        lax.fori_loop(0, K2 // 16, node16, 0)
        full = row0 + K2 <= N

        @pl.when(full)
        def _():
            pltpu.sync_copy(rs.at[0], o_hbm.at[c, pl.ds(row0, K2)])

        @pl.when(jnp.logical_and(jnp.logical_not(full), row0 < N))
        def _():
            pltpu.sync_copy(rs.at[0, pl.ds(0, 16)],
                            o_hbm.at[c, pl.ds(row0, 16)])


def kernel(x, src, tgt, W_f, b_f, W_w, b_w):
    src = src.astype(jnp.int32)
    tgt = tgt.astype(jnp.int32)
    W_w2 = W_w.reshape(1, 2 * D)
    xs0, xs1, xt0, xt1, ws2, wt2 = _dense(x, W_f, b_f, W_w2, b_w)
    ws1 = ws2[:, 0]
    wt1 = wt2[:, 0]
    a_e, mx = _edge_logits(src, tgt, ws1, wt1)
    o2 = _aggregate(src, tgt, a_e, xs0, xs1, xt0, xt1, mx)
    return jnp.concatenate([o2[0], o2[1]], axis=1)
